# baseline (speedup 1.0000x reference)
import functools

import jax
import jax.numpy as jnp
from jax import lax
from jax.experimental import pallas as pl
from jax.experimental.pallas import tpu as pltpu
from jax.experimental.pallas import tpu_sc as plsc

N = 10000
E = 320000
D = 128

C = 32
BLOCK = 6
NBLK0 = 104
NBLK1 = 2
CPT0 = BLOCK * NBLK0
CPT1 = BLOCK * NBLK1
NCH = 16 * (CPT0 + CPT1)
EPAD = NCH * C
PADN = 10240
RPT = PADN // 16


@functools.lru_cache(maxsize=1)
def _build_sc():
    mesh = plsc.VectorSubcoreMesh(core_axis_name="c", subcore_axis_name="s")

    @functools.partial(
        pl.kernel,
        mesh=mesh,
        out_type=[
            jax.ShapeDtypeStruct((2, PADN, D), jnp.float32),
            jax.ShapeDtypeStruct((2 * PADN,), jnp.float32),
        ],
        scratch_types=[
            pltpu.VMEM((2, BLOCK, 6, C), jnp.int32),
            pltpu.VMEM((C, D), jnp.float32),
            pltpu.VMEM((C, D), jnp.float32),
            pltpu.VMEM((C, D), jnp.float32),
            pltpu.VMEM((C, D), jnp.float32),
            pltpu.VMEM((C, D), jnp.float32),
            pltpu.VMEM((C, D), jnp.float32),
            pltpu.VMEM((C,), jnp.float32),
            pltpu.VMEM((128,), jnp.float32),
            pltpu.VMEM_SHARED((PADN, D), jnp.float32),
            pltpu.VMEM_SHARED((PADN,), jnp.float32),
            pltpu.SemaphoreType.DMA,
            pltpu.SemaphoreType.DMA,
            pltpu.SemaphoreType.DMA,
            pltpu.SemaphoreType.DMA,
            pltpu.SemaphoreType.DMA,
            pltpu.SemaphoreType.DMA,
            pltpu.SemaphoreType.DMA,
        ],
    )
    def sc_agg(nodes_hbm, rel_hbm, edges_hbm, out_acc, out_cnt,
               eidx, nb0, nb1, nb2, rb0, rb1, rb2, ones_v, z1,
               acc, cnt, gsem0, gsem1, gsem2, ssem0, ssem1, ssem2, isem):
        nbufs = (nb0, nb1, nb2)
        rbufs = (rb0, rb1, rb2)
        gsems = (gsem0, gsem1, gsem2)
        ssems = (ssem0, ssem1, ssem2)
        c = lax.axis_index("c")
        s = lax.axis_index("s")
        is0 = c == 0
        tile_chunk0 = jnp.where(is0, s * CPT0, 16 * CPT0 + s * CPT1)
        nblk_c = jnp.where(is0, NBLK0, NBLK1)
        cpt_c = nblk_c * BLOCK

        zeros16 = jnp.zeros((16,), jnp.float32)
        ones16 = jnp.ones((16,), jnp.float32)
        for t in range(C // 16):
            ones_v[pl.ds(t * 16, 16)] = ones16
        for t in range(128 // 16):
            z1[pl.ds(t * 16, 16)] = zeros16

        def zrow(i, carry):
            for t2 in range(D // 16):
                nb0[i, pl.ds(t2 * 16, 16)] = zeros16
            return carry
        lax.fori_loop(0, C, zrow, 0)

        row_base = s * RPT
        for i in range(RPT // C):
            pltpu.async_copy(nb0, acc.at[pl.ds(row_base + i * C, C)], gsem0)
        for r in range(5):
            pltpu.async_copy(z1, cnt.at[pl.ds(row_base + r * 128, 128)], gsem1)
        for i in range(RPT // C):
            pltpu.make_async_copy(
                nb0, acc.at[pl.ds(row_base + i * C, C)], gsem0).wait()
        for r in range(5):
            pltpu.make_async_copy(
                z1, cnt.at[pl.ds(row_base + r * 128, 128)], gsem1).wait()
        plsc.subcore_barrier()

        def issue_scatters(slot, idx_ref):
            pltpu.async_copy(nbufs[slot], acc.at[idx_ref], ssems[slot],
                             add=True)
            pltpu.async_copy(rbufs[slot], acc.at[idx_ref], ssems[slot],
                             add=True)
            pltpu.async_copy(ones_v, cnt.at[idx_ref], ssems[slot], add=True)

        def wait_scatters(slot, idx_ref):
            pltpu.make_async_copy(
                nbufs[slot], acc.at[idx_ref], ssems[slot]).wait()
            pltpu.make_async_copy(
                rbufs[slot], acc.at[idx_ref], ssems[slot]).wait()
            pltpu.make_async_copy(ones_v, cnt.at[idx_ref], ssems[slot]).wait()

        def issue_gathers(slot, src_ref, rel_ref):
            pltpu.async_copy(nodes_hbm.at[src_ref], nbufs[slot], gsems[slot])
            pltpu.async_copy(rel_hbm.at[rel_ref], rbufs[slot], gsems[slot])

        def wait_gathers(slot, src_ref, rel_ref):
            pltpu.make_async_copy(
                nodes_hbm.at[src_ref], nbufs[slot], gsems[slot]).wait()
            pltpu.make_async_copy(
                rel_hbm.at[rel_ref], rbufs[slot], gsems[slot]).wait()

        pltpu.sync_copy(edges_hbm.at[pl.ds(tile_chunk0, BLOCK)], eidx.at[0])
        issue_gathers(0, eidx.at[0, 0, 0], eidx.at[0, 0, 2])
        issue_gathers(1, eidx.at[0, 1, 0], eidx.at[0, 1, 2])

        def group_body(g, carry):
          for bb in range(2):
            blk = g * 2 + bb
            eb = bb
            en = 1 - bb
            for u in range(BLOCK):
                j = blk * BLOCK + u
                bj = (bb * BLOCK + u) % 3
                bn = (bj + 2) % 3
                wait_gathers(bj, eidx.at[eb, u, 0], eidx.at[eb, u, 2])
                issue_scatters(bj, eidx.at[eb, u, 4])

                @pl.when(j + 2 < cpt_c)
                def _():
                    @pl.when(j >= 1)
                    def _():
                        wait_scatters(bn, eidx.at[eb, u, 4])

                if u == 0:
                    @pl.when(blk + 1 < nblk_c)
                    def _():
                        pltpu.async_copy(
                            edges_hbm.at[
                                pl.ds(tile_chunk0 + (blk + 1) * BLOCK, BLOCK)],
                            eidx.at[en], isem)
                if u < BLOCK - 2:
                    @pl.when(j + 2 < cpt_c)
                    def _():
                        issue_gathers(bn, eidx.at[eb, u + 2, 0],
                                      eidx.at[eb, u + 2, 2])
                elif u == BLOCK - 2:
                    @pl.when(blk + 1 < nblk_c)
                    def _():
                        pltpu.make_async_copy(
                            edges_hbm.at[
                                pl.ds(tile_chunk0 + (blk + 1) * BLOCK, BLOCK)],
                            eidx.at[en], isem).wait()
                        issue_gathers(bn, eidx.at[en, 0, 0],
                                      eidx.at[en, 0, 2])
                else:
                    @pl.when(blk + 1 < nblk_c)
                    def _():
                        issue_gathers(bn, eidx.at[en, 1, 0],
                                      eidx.at[en, 1, 2])
          return carry

        lax.fori_loop(0, nblk_c // 2, group_body, 0)
        for k in range(3):
            wait_scatters(k, eidx.at[1, BLOCK - 3 + k, 4])
        plsc.subcore_barrier()

        for i in range(RPT // C):
            pltpu.async_copy(acc.at[pl.ds(row_base + i * C, C)],
                             out_acc.at[c, pl.ds(row_base + i * C, C)], gsem0)
        for r in range(5):
            pltpu.async_copy(
                cnt.at[pl.ds(row_base + r * 128, 128)],
                out_cnt.at[pl.ds(c * PADN + row_base + r * 128, 128)], gsem1)
        for i in range(RPT // C):
            pltpu.make_async_copy(
                acc.at[pl.ds(row_base + i * C, C)],
                out_acc.at[c, pl.ds(row_base + i * C, C)], gsem0).wait()
        for r in range(5):
            pltpu.make_async_copy(
                cnt.at[pl.ds(row_base + r * 128, 128)],
                out_cnt.at[pl.ds(c * PADN + row_base + r * 128, 128)],
                gsem1).wait()

    return sc_agg


def _combine_body(n_ref, a_ref, c_ref, wn_ref, ws_ref, we_ref, o_ref):
    nodes = n_ref[...]
    agg = a_ref[0] + a_ref[1]
    cnt = c_ref[0] + c_ref[1]
    is_dst = cnt > 0.0
    msg = jnp.dot(agg, wn_ref[...], preferred_element_type=jnp.float32)
    sl_s = jnp.dot(nodes, ws_ref[...], preferred_element_type=jnp.float32)
    sl_e = jnp.dot(nodes, we_ref[...], preferred_element_type=jnp.float32)
    o_ref[...] = nodes + msg + jnp.where(is_dst, sl_s, sl_e)


def _combine(nodes, acc2, cnt3, wn, ws, we):
    BLK = 400
    return pl.pallas_call(
        _combine_body,
        grid=(N // BLK,),
        in_specs=[
            pl.BlockSpec((BLK, D), lambda i: (i, 0)),
            pl.BlockSpec((2, BLK, D), lambda i: (0, i, 0)),
            pl.BlockSpec((2, BLK, 1), lambda i: (0, i, 0)),
            pl.BlockSpec((D, D), lambda i: (0, 0)),
            pl.BlockSpec((D, D), lambda i: (0, 0)),
            pl.BlockSpec((D, D), lambda i: (0, 0)),
        ],
        out_specs=pl.BlockSpec((BLK, D), lambda i: (i, 0)),
        out_shape=jax.ShapeDtypeStruct((N, D), jnp.float32),
    )(nodes, acc2, cnt3, wn, ws, we)


def kernel(nodes_embed, relation_embed, edges, w_neighbor, w_self,
           w_self_evolve):
    pad = EPAD - E
    zpad = jnp.zeros((pad,), jnp.int32)
    src = jnp.concatenate([edges[:, 0], zpad]).reshape(NCH, 1, C)
    rel = jnp.concatenate([edges[:, 1], zpad]).reshape(NCH, 1, C)
    dst = jnp.concatenate(
        [edges[:, 2], jnp.full((pad,), PADN - 1, jnp.int32)]).reshape(NCH, 1, C)
    zrow = jnp.zeros((NCH, 1, C), jnp.int32)
    packed = jnp.concatenate([src, zrow, rel, zrow, dst, zrow], axis=1)
    acc2, cnt2 = _build_sc()(nodes_embed, relation_embed, packed)
    cnt3 = cnt2.reshape(2, PADN, 1)
    return _combine(nodes_embed, acc2, cnt3,
                    w_neighbor, w_self, w_self_evolve)


# speedup vs baseline: 1.0767x; 1.0767x over previous
import functools

import jax
import jax.numpy as jnp
from jax import lax
from jax.experimental import pallas as pl
from jax.experimental.pallas import tpu as pltpu
from jax.experimental.pallas import tpu_sc as plsc

N = 10000
E = 320000
D = 128

C = 32
BLOCK = 6
NBLK0 = 92
NBLK1 = 14
CPT0 = BLOCK * NBLK0
CPT1 = BLOCK * NBLK1
NCH = 16 * (CPT0 + CPT1)
EPAD = NCH * C
PADN = 10240
RPT = PADN // 16


@functools.lru_cache(maxsize=1)
def _build_sc():
    mesh = plsc.VectorSubcoreMesh(core_axis_name="c", subcore_axis_name="s")

    @functools.partial(
        pl.kernel,
        mesh=mesh,
        out_type=[
            jax.ShapeDtypeStruct((2, PADN, D), jnp.float32),
            jax.ShapeDtypeStruct((2 * PADN,), jnp.float32),
        ],
        scratch_types=[
            pltpu.VMEM((2, BLOCK, 6, C), jnp.int32),
            pltpu.VMEM((C, D), jnp.float32),
            pltpu.VMEM((C, D), jnp.float32),
            pltpu.VMEM((C, D), jnp.float32),
            pltpu.VMEM((C, D), jnp.float32),
            pltpu.VMEM((C, D), jnp.float32),
            pltpu.VMEM((C, D), jnp.float32),
            pltpu.VMEM((C,), jnp.float32),
            pltpu.VMEM((128,), jnp.float32),
            pltpu.VMEM_SHARED((PADN, D), jnp.float32),
            pltpu.VMEM_SHARED((PADN,), jnp.float32),
            pltpu.SemaphoreType.DMA,
            pltpu.SemaphoreType.DMA,
            pltpu.SemaphoreType.DMA,
            pltpu.SemaphoreType.DMA,
            pltpu.SemaphoreType.DMA,
            pltpu.SemaphoreType.DMA,
            pltpu.SemaphoreType.DMA,
        ],
    )
    def sc_agg(nodes_hbm, rel_hbm, edges_hbm, out_acc, out_cnt,
               eidx, nb0, nb1, nb2, rb0, rb1, rb2, ones_v, z1,
               acc, cnt, gsem0, gsem1, gsem2, ssem0, ssem1, ssem2, isem):
        nbufs = (nb0, nb1, nb2)
        rbufs = (rb0, rb1, rb2)
        gsems = (gsem0, gsem1, gsem2)
        ssems = (ssem0, ssem1, ssem2)
        c = lax.axis_index("c")
        s = lax.axis_index("s")
        is0 = c == 0
        tile_chunk0 = jnp.where(is0, s * CPT0, 16 * CPT0 + s * CPT1)
        nblk_c = jnp.where(is0, NBLK0, NBLK1)
        cpt_c = nblk_c * BLOCK

        zeros16 = jnp.zeros((16,), jnp.float32)
        ones16 = jnp.ones((16,), jnp.float32)
        for t in range(C // 16):
            ones_v[pl.ds(t * 16, 16)] = ones16
        for t in range(128 // 16):
            z1[pl.ds(t * 16, 16)] = zeros16

        def zrow(i, carry):
            for t2 in range(D // 16):
                nb0[i, pl.ds(t2 * 16, 16)] = zeros16
            return carry
        lax.fori_loop(0, C, zrow, 0)

        row_base = s * RPT
        for i in range(RPT // C):
            pltpu.async_copy(nb0, acc.at[pl.ds(row_base + i * C, C)], gsem0)
        for r in range(5):
            pltpu.async_copy(z1, cnt.at[pl.ds(row_base + r * 128, 128)], gsem1)
        for i in range(RPT // C):
            pltpu.make_async_copy(
                nb0, acc.at[pl.ds(row_base + i * C, C)], gsem0).wait()
        for r in range(5):
            pltpu.make_async_copy(
                z1, cnt.at[pl.ds(row_base + r * 128, 128)], gsem1).wait()
        plsc.subcore_barrier()

        def issue_scatters(slot, idx_ref):
            pltpu.async_copy(nbufs[slot], acc.at[idx_ref], ssems[slot],
                             add=True)
            pltpu.async_copy(rbufs[slot], acc.at[idx_ref], ssems[slot],
                             add=True)
            pltpu.async_copy(ones_v, cnt.at[idx_ref], ssems[slot], add=True)

        def wait_scatters(slot, idx_ref):
            pltpu.make_async_copy(
                nbufs[slot], acc.at[idx_ref], ssems[slot]).wait()
            pltpu.make_async_copy(
                rbufs[slot], acc.at[idx_ref], ssems[slot]).wait()
            pltpu.make_async_copy(ones_v, cnt.at[idx_ref], ssems[slot]).wait()

        def issue_gathers(slot, src_ref, rel_ref):
            pltpu.async_copy(nodes_hbm.at[src_ref], nbufs[slot], gsems[slot])
            pltpu.async_copy(rel_hbm.at[rel_ref], rbufs[slot], gsems[slot])

        def wait_gathers(slot, src_ref, rel_ref):
            pltpu.make_async_copy(
                nodes_hbm.at[src_ref], nbufs[slot], gsems[slot]).wait()
            pltpu.make_async_copy(
                rel_hbm.at[rel_ref], rbufs[slot], gsems[slot]).wait()

        pltpu.sync_copy(edges_hbm.at[pl.ds(tile_chunk0, BLOCK)], eidx.at[0])
        issue_gathers(0, eidx.at[0, 0, 0], eidx.at[0, 0, 2])
        issue_gathers(1, eidx.at[0, 1, 0], eidx.at[0, 1, 2])

        def group_body(g, carry):
          for bb in range(2):
            blk = g * 2 + bb
            eb = bb
            en = 1 - bb
            for u in range(BLOCK):
                j = blk * BLOCK + u
                bj = (bb * BLOCK + u) % 3
                bn = (bj + 2) % 3
                wait_gathers(bj, eidx.at[eb, u, 0], eidx.at[eb, u, 2])
                issue_scatters(bj, eidx.at[eb, u, 4])

                @pl.when(j + 2 < cpt_c)
                def _():
                    @pl.when(j >= 1)
                    def _():
                        wait_scatters(bn, eidx.at[eb, u, 4])

                if u == 0:
                    @pl.when(blk + 1 < nblk_c)
                    def _():
                        pltpu.async_copy(
                            edges_hbm.at[
                                pl.ds(tile_chunk0 + (blk + 1) * BLOCK, BLOCK)],
                            eidx.at[en], isem)
                if u < BLOCK - 2:
                    @pl.when(j + 2 < cpt_c)
                    def _():
                        issue_gathers(bn, eidx.at[eb, u + 2, 0],
                                      eidx.at[eb, u + 2, 2])
                elif u == BLOCK - 2:
                    @pl.when(blk + 1 < nblk_c)
                    def _():
                        pltpu.make_async_copy(
                            edges_hbm.at[
                                pl.ds(tile_chunk0 + (blk + 1) * BLOCK, BLOCK)],
                            eidx.at[en], isem).wait()
                        issue_gathers(bn, eidx.at[en, 0, 0],
                                      eidx.at[en, 0, 2])
                else:
                    @pl.when(blk + 1 < nblk_c)
                    def _():
                        issue_gathers(bn, eidx.at[en, 1, 0],
                                      eidx.at[en, 1, 2])
          return carry

        lax.fori_loop(0, nblk_c // 2, group_body, 0)
        for k in range(3):
            wait_scatters(k, eidx.at[1, BLOCK - 3 + k, 4])
        plsc.subcore_barrier()

        pltpu.async_copy(acc.at[pl.ds(row_base, RPT)],
                         out_acc.at[c, pl.ds(row_base, RPT)], gsem0)
        pltpu.async_copy(cnt.at[pl.ds(row_base, RPT)],
                         out_cnt.at[pl.ds(c * PADN + row_base, RPT)], gsem1)
        pltpu.make_async_copy(acc.at[pl.ds(row_base, RPT)],
                              out_acc.at[c, pl.ds(row_base, RPT)],
                              gsem0).wait()
        pltpu.make_async_copy(cnt.at[pl.ds(row_base, RPT)],
                              out_cnt.at[pl.ds(c * PADN + row_base, RPT)],
                              gsem1).wait()

    return sc_agg


def _combine_body(n_ref, a_ref, c_ref, wn_ref, ws_ref, we_ref, o_ref):
    nodes = n_ref[...]
    agg = a_ref[0] + a_ref[1]
    cnt = c_ref[0] + c_ref[1]
    is_dst = cnt > 0.0
    msg = jnp.dot(agg, wn_ref[...], preferred_element_type=jnp.float32)
    sl_s = jnp.dot(nodes, ws_ref[...], preferred_element_type=jnp.float32)
    sl_e = jnp.dot(nodes, we_ref[...], preferred_element_type=jnp.float32)
    o_ref[...] = nodes + msg + jnp.where(is_dst, sl_s, sl_e)


def _combine(nodes, acc2, cnt3, wn, ws, we):
    BLK = 400
    return pl.pallas_call(
        _combine_body,
        grid=(N // BLK,),
        in_specs=[
            pl.BlockSpec((BLK, D), lambda i: (i, 0)),
            pl.BlockSpec((2, BLK, D), lambda i: (0, i, 0)),
            pl.BlockSpec((2, BLK, 1), lambda i: (0, i, 0)),
            pl.BlockSpec((D, D), lambda i: (0, 0)),
            pl.BlockSpec((D, D), lambda i: (0, 0)),
            pl.BlockSpec((D, D), lambda i: (0, 0)),
        ],
        out_specs=pl.BlockSpec((BLK, D), lambda i: (i, 0)),
        out_shape=jax.ShapeDtypeStruct((N, D), jnp.float32),
    )(nodes, acc2, cnt3, wn, ws, we)


def kernel(nodes_embed, relation_embed, edges, w_neighbor, w_self,
           w_self_evolve):
    pad = EPAD - E
    zpad = jnp.zeros((pad,), jnp.int32)
    src = jnp.concatenate([edges[:, 0], zpad]).reshape(NCH, 1, C)
    rel = jnp.concatenate([edges[:, 1], zpad]).reshape(NCH, 1, C)
    dst = jnp.concatenate(
        [edges[:, 2], jnp.full((pad,), PADN - 1, jnp.int32)]).reshape(NCH, 1, C)
    zrow = jnp.zeros((NCH, 1, C), jnp.int32)
    packed = jnp.concatenate([src, zrow, rel, zrow, dst, zrow], axis=1)
    acc2, cnt2 = _build_sc()(nodes_embed, relation_embed, packed)
    cnt3 = cnt2.reshape(2, PADN, 1)
    return _combine(nodes_embed, acc2, cnt3,
                    w_neighbor, w_self, w_self_evolve)


# speedup vs baseline: 1.3116x; 1.2181x over previous
import functools

import jax
import jax.numpy as jnp
from jax import lax
from jax.experimental import pallas as pl
from jax.experimental.pallas import tpu as pltpu
from jax.experimental.pallas import tpu_sc as plsc

N = 10000
E = 320000
D = 128

C = 32
BLOCK = 6
NBLK0 = 97
NBLK1 = 9
CPT0 = BLOCK * NBLK0
CPT1 = BLOCK * NBLK1
NCH = 16 * (CPT0 + CPT1)
EPAD = NCH * C
PADN = 10240
RPT = PADN // 16


@functools.lru_cache(maxsize=1)
def _build_sc():
    mesh = plsc.VectorSubcoreMesh(core_axis_name="c", subcore_axis_name="s")

    @functools.partial(
        pl.kernel,
        mesh=mesh,
        out_type=[
            jax.ShapeDtypeStruct((2, PADN, D), jnp.float32),
            jax.ShapeDtypeStruct((2 * PADN,), jnp.float32),
        ],
        scratch_types=[
            pltpu.VMEM((2, BLOCK, 6, C), jnp.int32),
            pltpu.VMEM((3, C, D), jnp.float32),
            pltpu.VMEM((3, C, D), jnp.float32),
            pltpu.VMEM((C,), jnp.float32),
            pltpu.VMEM((128,), jnp.float32),
            pltpu.VMEM_SHARED((PADN, D), jnp.float32),
            pltpu.VMEM_SHARED((PADN,), jnp.float32),
            pltpu.SemaphoreType.DMA((3,)),
            pltpu.SemaphoreType.DMA((3,)),
            pltpu.SemaphoreType.DMA,
        ],
    )
    def sc_agg(nodes_hbm, rel_hbm, edges_hbm, out_acc, out_cnt,
               eidx, nbuf, rbuf, ones_v, z1,
               acc, cnt, gsem, ssem, isem):
        c = lax.axis_index("c")
        s = lax.axis_index("s")
        is0 = c == 0
        tile_chunk0 = jnp.where(is0, s * CPT0, 16 * CPT0 + s * CPT1)
        nblk_c = jnp.where(is0, NBLK0, NBLK1)
        cpt_c = nblk_c * BLOCK

        zeros16 = jnp.zeros((16,), jnp.float32)
        ones16 = jnp.ones((16,), jnp.float32)
        for t in range(C // 16):
            ones_v[pl.ds(t * 16, 16)] = ones16
        for t in range(128 // 16):
            z1[pl.ds(t * 16, 16)] = zeros16

        def zrow(i, carry):
            for t2 in range(D // 16):
                nbuf[0, i, pl.ds(t2 * 16, 16)] = zeros16
            return carry
        lax.fori_loop(0, C, zrow, 0)

        row_base = s * RPT
        for i in range(RPT // C):
            pltpu.async_copy(nbuf.at[0], acc.at[pl.ds(row_base + i * C, C)],
                             gsem.at[0])
        for r in range(5):
            pltpu.async_copy(z1, cnt.at[pl.ds(row_base + r * 128, 128)],
                             gsem.at[1])
        for i in range(RPT // C):
            pltpu.make_async_copy(
                nbuf.at[0], acc.at[pl.ds(row_base + i * C, C)],
                gsem.at[0]).wait()
        for r in range(5):
            pltpu.make_async_copy(
                z1, cnt.at[pl.ds(row_base + r * 128, 128)], gsem.at[1]).wait()
        plsc.subcore_barrier()

        def issue_scatters(b, idx_ref):
            pltpu.async_copy(nbuf.at[b], acc.at[idx_ref], ssem.at[b],
                             add=True)
            pltpu.async_copy(rbuf.at[b], acc.at[idx_ref], ssem.at[b],
                             add=True)
            pltpu.async_copy(ones_v, cnt.at[idx_ref], ssem.at[b], add=True)

        def wait_scatters(b, idx_ref):
            pltpu.make_async_copy(
                nbuf.at[b], acc.at[idx_ref], ssem.at[b]).wait()
            pltpu.make_async_copy(
                rbuf.at[b], acc.at[idx_ref], ssem.at[b]).wait()
            pltpu.make_async_copy(ones_v, cnt.at[idx_ref], ssem.at[b]).wait()

        def issue_gathers(b, src_ref, rel_ref):
            pltpu.async_copy(nodes_hbm.at[src_ref], nbuf.at[b], gsem.at[b])
            pltpu.async_copy(rel_hbm.at[rel_ref], rbuf.at[b], gsem.at[b])

        def wait_gathers(b, src_ref, rel_ref):
            pltpu.make_async_copy(
                nodes_hbm.at[src_ref], nbuf.at[b], gsem.at[b]).wait()
            pltpu.make_async_copy(
                rel_hbm.at[rel_ref], rbuf.at[b], gsem.at[b]).wait()

        pltpu.sync_copy(edges_hbm.at[pl.ds(tile_chunk0, BLOCK)], eidx.at[0])

        @pl.when(nblk_c > 1)
        def _():
            pltpu.async_copy(edges_hbm.at[pl.ds(tile_chunk0 + BLOCK, BLOCK)],
                             eidx.at[1], isem)
        issue_gathers(0, eidx.at[0, 0, 0], eidx.at[0, 0, 2])
        issue_gathers(1, eidx.at[0, 1, 0], eidx.at[0, 1, 2])

        def chunk_body(j, carry):
            b = j % 3
            bn = (j + 2) % 3
            blk = j // BLOCK
            u = j % BLOCK
            eb = blk % 2
            wait_gathers(b, eidx.at[eb, u, 0], eidx.at[eb, u, 2])
            issue_scatters(b, eidx.at[eb, u, 4])

            @pl.when((j >= 1) & (j + 2 < cpt_c))
            def _():
                wait_scatters(bn, eidx.at[eb, u, 4])

            @pl.when((u == 0) & (j >= 1) & (blk + 1 < nblk_c))
            def _():
                pltpu.async_copy(
                    edges_hbm.at[pl.ds(tile_chunk0 + (blk + 1) * BLOCK,
                                       BLOCK)],
                    eidx.at[1 - eb], isem)

            @pl.when((u == BLOCK - 2) & (blk + 1 < nblk_c))
            def _():
                pltpu.make_async_copy(
                    edges_hbm.at[pl.ds(tile_chunk0 + (blk + 1) * BLOCK,
                                       BLOCK)],
                    eidx.at[1 - eb], isem).wait()

            @pl.when(j + 2 < cpt_c)
            def _():
                j2 = j + 2
                e2 = (j2 // BLOCK) % 2
                u2 = j2 % BLOCK
                issue_gathers(bn, eidx.at[e2, u2, 0], eidx.at[e2, u2, 2])
            return carry

        lax.fori_loop(0, cpt_c, chunk_body, 0)
        for k in range(3):
            wait_scatters(k, eidx.at[0, 0, 4])
        plsc.subcore_barrier()

        pltpu.async_copy(acc.at[pl.ds(row_base, RPT)],
                         out_acc.at[c, pl.ds(row_base, RPT)], gsem.at[0])
        pltpu.async_copy(cnt.at[pl.ds(row_base, RPT)],
                         out_cnt.at[pl.ds(c * PADN + row_base, RPT)],
                         gsem.at[1])
        pltpu.make_async_copy(acc.at[pl.ds(row_base, RPT)],
                              out_acc.at[c, pl.ds(row_base, RPT)],
                              gsem.at[0]).wait()
        pltpu.make_async_copy(cnt.at[pl.ds(row_base, RPT)],
                              out_cnt.at[pl.ds(c * PADN + row_base, RPT)],
                              gsem.at[1]).wait()

    return sc_agg


def _combine_body(n_ref, a_ref, c_ref, wn_ref, ws_ref, we_ref, o_ref):
    nodes = n_ref[...]
    agg = a_ref[0] + a_ref[1]
    cnt = c_ref[0] + c_ref[1]
    is_dst = cnt > 0.0
    msg = jnp.dot(agg, wn_ref[...], preferred_element_type=jnp.float32)
    sl_s = jnp.dot(nodes, ws_ref[...], preferred_element_type=jnp.float32)
    sl_e = jnp.dot(nodes, we_ref[...], preferred_element_type=jnp.float32)
    o_ref[...] = nodes + msg + jnp.where(is_dst, sl_s, sl_e)


def _combine(nodes, acc2, cnt3, wn, ws, we):
    BLK = 400
    return pl.pallas_call(
        _combine_body,
        grid=(N // BLK,),
        in_specs=[
            pl.BlockSpec((BLK, D), lambda i: (i, 0)),
            pl.BlockSpec((2, BLK, D), lambda i: (0, i, 0)),
            pl.BlockSpec((2, BLK, 1), lambda i: (0, i, 0)),
            pl.BlockSpec((D, D), lambda i: (0, 0)),
            pl.BlockSpec((D, D), lambda i: (0, 0)),
            pl.BlockSpec((D, D), lambda i: (0, 0)),
        ],
        out_specs=pl.BlockSpec((BLK, D), lambda i: (i, 0)),
        out_shape=jax.ShapeDtypeStruct((N, D), jnp.float32),
    )(nodes, acc2, cnt3, wn, ws, we)


def kernel(nodes_embed, relation_embed, edges, w_neighbor, w_self,
           w_self_evolve):
    pad = EPAD - E
    zpad = jnp.zeros((pad,), jnp.int32)
    src = jnp.concatenate([edges[:, 0], zpad]).reshape(NCH, 1, C)
    rel = jnp.concatenate([edges[:, 1], zpad]).reshape(NCH, 1, C)
    dst = jnp.concatenate(
        [edges[:, 2], jnp.full((pad,), PADN - 1, jnp.int32)]).reshape(NCH, 1, C)
    zrow = jnp.zeros((NCH, 1, C), jnp.int32)
    packed = jnp.concatenate([src, zrow, rel, zrow, dst, zrow], axis=1)
    acc2, cnt2 = _build_sc()(nodes_embed, relation_embed, packed)
    cnt3 = cnt2.reshape(2, PADN, 1)
    return _combine(nodes_embed, acc2, cnt3,
                    w_neighbor, w_self, w_self_evolve)


# speedup vs baseline: 1.4829x; 1.1307x over previous
import functools

import jax
import jax.numpy as jnp
from jax import lax
from jax.experimental import pallas as pl
from jax.experimental.pallas import tpu as pltpu
from jax.experimental.pallas import tpu_sc as plsc

N = 10000
E = 320000
D = 128

C = 32
BLOCK = 6
NBLK0 = 97
NBLK1 = 9
CPT0 = BLOCK * NBLK0
CPT1 = BLOCK * NBLK1
NCH = 16 * (CPT0 + CPT1)
EPAD = NCH * C
PADN = 10240
BC = BLOCK * C
RPT = PADN // 16


@functools.lru_cache(maxsize=1)
def _build_sc():
    mesh = plsc.VectorSubcoreMesh(core_axis_name="c", subcore_axis_name="s")

    @functools.partial(
        pl.kernel,
        mesh=mesh,
        out_type=[
            jax.ShapeDtypeStruct((2, PADN, D), jnp.float32),
            jax.ShapeDtypeStruct((2 * PADN,), jnp.float32),
        ],
        scratch_types=[
            pltpu.VMEM((2 * 3 * BLOCK * C,), jnp.int32),
            pltpu.VMEM((3, C, D), jnp.float32),
            pltpu.VMEM((3, C, D), jnp.float32),
            pltpu.VMEM((C,), jnp.float32),
            pltpu.VMEM((128,), jnp.float32),
            pltpu.VMEM_SHARED((PADN, D), jnp.float32),
            pltpu.VMEM_SHARED((PADN,), jnp.float32),
            pltpu.SemaphoreType.DMA((3,)),
            pltpu.SemaphoreType.DMA((3,)),
            pltpu.SemaphoreType.DMA,
        ],
    )
    def sc_agg(nodes_hbm, rel_hbm, edges_hbm, out_acc, out_cnt,
               eidx, nbuf, rbuf, ones_v, z1,
               acc, cnt, gsem, ssem, isem):
        c = lax.axis_index("c")
        s = lax.axis_index("s")
        is0 = c == 0
        tile_chunk0 = jnp.where(is0, s * CPT0, 16 * CPT0 + s * CPT1)
        nblk_c = jnp.where(is0, NBLK0, NBLK1)
        cpt_c = nblk_c * BLOCK

        zeros16 = jnp.zeros((16,), jnp.float32)
        ones16 = jnp.ones((16,), jnp.float32)
        for t in range(C // 16):
            ones_v[pl.ds(t * 16, 16)] = ones16
        for t in range(128 // 16):
            z1[pl.ds(t * 16, 16)] = zeros16

        def zrow(i, carry):
            for t2 in range(D // 16):
                nbuf[0, i, pl.ds(t2 * 16, 16)] = zeros16
            return carry
        lax.fori_loop(0, C, zrow, 0)

        row_base = s * RPT
        for i in range(RPT // C):
            pltpu.async_copy(nbuf.at[0], acc.at[pl.ds(row_base + i * C, C)],
                             gsem.at[0])
        for r in range(5):
            pltpu.async_copy(z1, cnt.at[pl.ds(row_base + r * 128, 128)],
                             gsem.at[1])
        for i in range(RPT // C):
            pltpu.make_async_copy(
                nbuf.at[0], acc.at[pl.ds(row_base + i * C, C)],
                gsem.at[0]).wait()
        for r in range(5):
            pltpu.make_async_copy(
                z1, cnt.at[pl.ds(row_base + r * 128, 128)], gsem.at[1]).wait()
        plsc.subcore_barrier()

        def issue_scatters(b, idx_ref):
            pltpu.async_copy(nbuf.at[b], acc.at[idx_ref], ssem.at[b],
                             add=True)
            pltpu.async_copy(rbuf.at[b], acc.at[idx_ref], ssem.at[b],
                             add=True)
            pltpu.async_copy(ones_v, cnt.at[idx_ref], ssem.at[b], add=True)

        def wait_scatters(b, idx_ref):
            pltpu.make_async_copy(
                nbuf.at[b], acc.at[idx_ref], ssem.at[b]).wait()
            pltpu.make_async_copy(
                rbuf.at[b], acc.at[idx_ref], ssem.at[b]).wait()
            pltpu.make_async_copy(ones_v, cnt.at[idx_ref], ssem.at[b]).wait()

        def issue_gathers(b, src_ref, rel_ref):
            pltpu.async_copy(nodes_hbm.at[src_ref], nbuf.at[b], gsem.at[b])
            pltpu.async_copy(rel_hbm.at[rel_ref], rbuf.at[b], gsem.at[b])

        def wait_gathers(b, src_ref, rel_ref):
            pltpu.make_async_copy(
                nodes_hbm.at[src_ref], nbuf.at[b], gsem.at[b]).wait()
            pltpu.make_async_copy(
                rel_hbm.at[rel_ref], rbuf.at[b], gsem.at[b]).wait()

        def load_idx_block(blk, slot, sem):
            base = (tile_chunk0 + blk * BLOCK) * C
            for t in range(3):
                pltpu.async_copy(
                    edges_hbm.at[pl.ds(t * EPAD + base, BC)],
                    eidx.at[pl.ds(slot * 3 * BC + t * BC, BC)], sem)

        def wait_idx_block(blk, slot, sem):
            base = (tile_chunk0 + blk * BLOCK) * C
            for t in range(3):
                pltpu.make_async_copy(
                    edges_hbm.at[pl.ds(t * EPAD + base, BC)],
                    eidx.at[pl.ds(slot * 3 * BC + t * BC, BC)], sem).wait()

        def idx_ref(eb, u, t):
            return eidx.at[pl.ds(eb * 3 * BC + t * BC + u * C, C)]

        load_idx_block(0, 0, isem)
        wait_idx_block(0, 0, isem)

        @pl.when(nblk_c > 1)
        def _():
            load_idx_block(1, 1, isem)
        issue_gathers(0, idx_ref(0, 0, 0), idx_ref(0, 0, 1))
        issue_gathers(1, idx_ref(0, 1, 0), idx_ref(0, 1, 1))

        def chunk_body(j, carry):
            b = j % 3
            bn = (j + 2) % 3
            blk = j // BLOCK
            u = j % BLOCK
            eb = blk % 2
            wait_gathers(b, idx_ref(eb, u, 0), idx_ref(eb, u, 1))
            issue_scatters(b, idx_ref(eb, u, 2))

            @pl.when((j >= 1) & (j + 2 < cpt_c))
            def _():
                wait_scatters(bn, idx_ref(eb, u, 2))

            @pl.when((u == 0) & (j >= 1) & (blk + 1 < nblk_c))
            def _():
                load_idx_block(blk + 1, 1 - eb, isem)

            @pl.when((u == BLOCK - 2) & (blk + 1 < nblk_c))
            def _():
                wait_idx_block(blk + 1, 1 - eb, isem)

            @pl.when(j + 2 < cpt_c)
            def _():
                j2 = j + 2
                e2 = (j2 // BLOCK) % 2
                u2 = j2 % BLOCK
                issue_gathers(bn, idx_ref(e2, u2, 0), idx_ref(e2, u2, 1))
            return carry

        lax.fori_loop(0, cpt_c, chunk_body, 0)
        for k in range(3):
            wait_scatters(k, idx_ref(0, 0, 2))
        plsc.subcore_barrier()

        pltpu.async_copy(acc.at[pl.ds(row_base, RPT)],
                         out_acc.at[c, pl.ds(row_base, RPT)], gsem.at[0])
        pltpu.async_copy(cnt.at[pl.ds(row_base, RPT)],
                         out_cnt.at[pl.ds(c * PADN + row_base, RPT)],
                         gsem.at[1])
        pltpu.make_async_copy(acc.at[pl.ds(row_base, RPT)],
                              out_acc.at[c, pl.ds(row_base, RPT)],
                              gsem.at[0]).wait()
        pltpu.make_async_copy(cnt.at[pl.ds(row_base, RPT)],
                              out_cnt.at[pl.ds(c * PADN + row_base, RPT)],
                              gsem.at[1]).wait()

    return sc_agg


def _combine_body(n_ref, a_ref, c_ref, wn_ref, ws_ref, we_ref, o_ref):
    nodes = n_ref[...]
    agg = a_ref[0] + a_ref[1]
    cnt = c_ref[0] + c_ref[1]
    is_dst = cnt > 0.0
    msg = jnp.dot(agg, wn_ref[...], preferred_element_type=jnp.float32)
    sl_s = jnp.dot(nodes, ws_ref[...], preferred_element_type=jnp.float32)
    sl_e = jnp.dot(nodes, we_ref[...], preferred_element_type=jnp.float32)
    o_ref[...] = nodes + msg + jnp.where(is_dst, sl_s, sl_e)


def _combine(nodes, acc2, cnt3, wn, ws, we):
    BLK = 400
    return pl.pallas_call(
        _combine_body,
        grid=(N // BLK,),
        in_specs=[
            pl.BlockSpec((BLK, D), lambda i: (i, 0)),
            pl.BlockSpec((2, BLK, D), lambda i: (0, i, 0)),
            pl.BlockSpec((2, BLK, 1), lambda i: (0, i, 0)),
            pl.BlockSpec((D, D), lambda i: (0, 0)),
            pl.BlockSpec((D, D), lambda i: (0, 0)),
            pl.BlockSpec((D, D), lambda i: (0, 0)),
        ],
        out_specs=pl.BlockSpec((BLK, D), lambda i: (i, 0)),
        out_shape=jax.ShapeDtypeStruct((N, D), jnp.float32),
    )(nodes, acc2, cnt3, wn, ws, we)


def kernel(nodes_embed, relation_embed, edges, w_neighbor, w_self,
           w_self_evolve):
    pad = EPAD - E
    zpad = jnp.zeros((pad,), jnp.int32)
    packed = jnp.concatenate([
        edges[:, 0], zpad,
        edges[:, 1], zpad,
        edges[:, 2], jnp.full((pad,), PADN - 1, jnp.int32),
    ])
    acc2, cnt2 = _build_sc()(nodes_embed, relation_embed, packed)
    cnt3 = cnt2.reshape(2, PADN, 1)
    return _combine(nodes_embed, acc2, cnt3,
                    w_neighbor, w_self, w_self_evolve)
